# baseline (speedup 1.0000x reference)
import functools

import jax
import jax.numpy as jnp
from jax import lax
from jax.experimental import pallas as pl
from jax.experimental.pallas import tpu as pltpu
from jax.experimental.pallas import tpu_sc as plsc

_INFO = plsc.get_sparse_core_info()
_NC = _INFO.num_cores
_NS = _INFO.num_subcores
_NW = _NC * _NS


_NBUF = 3


@functools.lru_cache(maxsize=None)
def _make_gather(B, D, C):
  assert B % (_NW * C) == 0 and C % 8 == 0 and C <= 128
  per_worker = B // _NW
  n_chunks = per_worker // C
  assert n_chunks >= _NBUF
  mesh = plsc.VectorSubcoreMesh(core_axis_name="c", subcore_axis_name="s")

  @functools.partial(
      pl.kernel,
      out_type=jax.ShapeDtypeStruct((B, D), jnp.float32),
      mesh=mesh,
      scratch_types=[
          pltpu.VMEM((n_chunks, C), jnp.int32),
      ] + [pltpu.VMEM((C, D), jnp.float32)] * _NBUF
        + [pltpu.SemaphoreType.DMA] * (2 * _NBUF),
  )
  def gather(ids_hbm, table_hbm, out_hbm, idx_v, *bufs):
    rows = bufs[:_NBUF]
    gsem = bufs[_NBUF:2 * _NBUF]
    ssem = bufs[2 * _NBUF:]
    wid = lax.axis_index("s") * _NC + lax.axis_index("c")
    base = wid * per_worker

    pltpu.sync_copy(ids_hbm.at[pl.ds(wid * n_chunks, n_chunks)], idx_v)

    def fire_gather(j):
      b = j % _NBUF
      return pltpu.async_copy(table_hbm.at[idx_v.at[j]], rows[b], gsem[b])

    def fire_store(j):
      b = j % _NBUF
      return pltpu.async_copy(rows[b], out_hbm.at[pl.ds(base + j * C, C)],
                              ssem[b])

    gd = [None] * _NBUF
    sd = [None] * _NBUF
    for j in range(_NBUF - 1):
      gd[j] = fire_gather(j)
    for j in range(n_chunks):
      b = j % _NBUF
      gd[b].wait()
      sd[b] = fire_store(j)
      if j + 2 < n_chunks:
        nb = (j + 2) % _NBUF
        if sd[nb] is not None:
          sd[nb].wait()
        gd[nb] = fire_gather(j + 2)
    for j in range(n_chunks - _NBUF, n_chunks):
      if j >= 0:
        sd[j % _NBUF].wait()

  return gather


def kernel(position_ids, table):
  batch, seq = position_ids.shape
  d = table.shape[1]
  C = 32
  ids = position_ids.reshape(-1, C).astype(jnp.int32)
  out = _make_gather(batch * seq, d, C)(ids, table)
  return out.reshape(batch, seq, d)


# speedup vs baseline: 1.0144x; 1.0144x over previous
import functools

import jax
import jax.numpy as jnp
from jax import lax
from jax.experimental import pallas as pl
from jax.experimental.pallas import tpu as pltpu
from jax.experimental.pallas import tpu_sc as plsc

_INFO = plsc.get_sparse_core_info()
_NC = _INFO.num_cores
_NS = _INFO.num_subcores
_NW = _NC * _NS


def _chunk_sizes(per_worker, a, b):
  sizes = []
  left = per_worker
  while left > 0:
    want = a if len(sizes) % 2 == 0 else b
    sizes.append(min(want, left))
    left -= sizes[-1]
  return sizes


@functools.lru_cache(maxsize=None)
def _make_gather(B, D):
  assert B % _NW == 0
  per_worker = B // _NW
  CA, CB = 64, 56
  sizes = _chunk_sizes(per_worker, CA, CB)
  offs = [sum(sizes[:j]) for j in range(len(sizes))]
  n_chunks = len(sizes)
  assert all(s % 8 == 0 and s <= 128 for s in sizes)
  mesh = plsc.VectorSubcoreMesh(core_axis_name="c", subcore_axis_name="s")

  @functools.partial(
      pl.kernel,
      out_type=jax.ShapeDtypeStruct((B, D), jnp.float32),
      mesh=mesh,
      scratch_types=[
          pltpu.VMEM((per_worker,), jnp.int32),
          pltpu.VMEM((CA, D), jnp.float32),
          pltpu.VMEM((CB, D), jnp.float32),
      ] + [pltpu.SemaphoreType.DMA] * 4,
  )
  def gather(ids_hbm, table_hbm, out_hbm, idx_v, rows_a, rows_b, g0, g1, s0,
             s1):
    rows = (rows_a, rows_b)
    gsem = (g0, g1)
    ssem = (s0, s1)
    wid = lax.axis_index("s") * _NC + lax.axis_index("c")
    base = wid * per_worker

    pltpu.sync_copy(ids_hbm.at[pl.ds(base, per_worker)], idx_v)

    def fire_gather(j):
      b = j % 2
      dst = rows[b] if sizes[j] == rows[b].shape[0] else rows[b].at[
          pl.ds(0, sizes[j])]
      return pltpu.async_copy(
          table_hbm.at[idx_v.at[pl.ds(offs[j], sizes[j])]], dst, gsem[b])

    def fire_store(j):
      b = j % 2
      src = rows[b] if sizes[j] == rows[b].shape[0] else rows[b].at[
          pl.ds(0, sizes[j])]
      return pltpu.async_copy(src, out_hbm.at[pl.ds(base + offs[j], sizes[j])],
                              ssem[b])

    gd = [None, None]
    sd = [None, None]
    gd[0] = fire_gather(0)
    if n_chunks > 1:
      gd[1] = fire_gather(1)
    for j in range(n_chunks):
      b = j % 2
      gd[b].wait()
      sd[b] = fire_store(j)
      if j + 2 < n_chunks:
        sd[b].wait()
        gd[b] = fire_gather(j + 2)
    for j in range(max(0, n_chunks - 2), n_chunks):
      sd[j % 2].wait()

  return gather


def kernel(position_ids, table):
  batch, seq = position_ids.shape
  d = table.shape[1]
  ids = position_ids.reshape(-1).astype(jnp.int32)
  out = _make_gather(batch * seq, d)(ids, table)
  return out.reshape(batch, seq, d)


# speedup vs baseline: 1.0226x; 1.0081x over previous
import functools

import jax
import jax.numpy as jnp
from jax import lax
from jax.experimental import pallas as pl
from jax.experimental.pallas import tpu as pltpu
from jax.experimental.pallas import tpu_sc as plsc

_INFO = plsc.get_sparse_core_info()
_NC = _INFO.num_cores
_NS = _INFO.num_subcores
_NW = _NC * _NS


def _chunk_sizes(per_worker, a, b):
  sizes = []
  left = per_worker
  while left > 0:
    want = a if len(sizes) % 2 == 0 else b
    sizes.append(min(want, left))
    left -= sizes[-1]
  return sizes


@functools.lru_cache(maxsize=None)
def _make_gather(B, D):
  assert B % _NW == 0
  per_worker = B // _NW
  CA, CB = 64, 56
  sizes = _chunk_sizes(per_worker, CA, CB)
  offs = [sum(sizes[:j]) for j in range(len(sizes))]
  n_chunks = len(sizes)
  assert all(s % 8 == 0 and s <= 128 for s in sizes)
  mesh = plsc.VectorSubcoreMesh(core_axis_name="c", subcore_axis_name="s")

  @functools.partial(
      pl.kernel,
      out_type=jax.ShapeDtypeStruct((B, D), jnp.float32),
      mesh=mesh,
      scratch_types=[
          pltpu.VMEM((per_worker,), jnp.int32),
          pltpu.VMEM((CA, D), jnp.float32),
          pltpu.VMEM((CB, D), jnp.float32),
      ] + [pltpu.SemaphoreType.DMA] * 4,
  )
  def gather(ids_hbm, table_hbm, out_hbm, idx_v, rows_a, rows_b, g0, g1, s0,
             s1):
    rows = (rows_a, rows_b)
    gsem = (g0, g1)
    ssem = (s0, s1)
    wid = lax.axis_index("s") * _NC + lax.axis_index("c")
    base = wid * per_worker

    head = sizes[0]
    pltpu.sync_copy(ids_hbm.at[pl.ds(base, head)], idx_v.at[pl.ds(0, head)])

    def fire_gather(j):
      b = j % 2
      dst = rows[b] if sizes[j] == rows[b].shape[0] else rows[b].at[
          pl.ds(0, sizes[j])]
      return pltpu.async_copy(
          table_hbm.at[idx_v.at[pl.ds(offs[j], sizes[j])]], dst, gsem[b])

    def fire_store(j):
      b = j % 2
      src = rows[b] if sizes[j] == rows[b].shape[0] else rows[b].at[
          pl.ds(0, sizes[j])]
      return pltpu.async_copy(src, out_hbm.at[pl.ds(base + offs[j], sizes[j])],
                              ssem[b])

    gd = [None, None]
    sd = [None, None]
    gd[0] = fire_gather(0)
    if n_chunks > 1:
      pltpu.sync_copy(ids_hbm.at[pl.ds(base + head, per_worker - head)],
                      idx_v.at[pl.ds(head, per_worker - head)])
      gd[1] = fire_gather(1)
    for j in range(n_chunks):
      b = j % 2
      gd[b].wait()
      sd[b] = fire_store(j)
      if j + 2 < n_chunks:
        sd[b].wait()
        gd[b] = fire_gather(j + 2)
    for j in range(max(0, n_chunks - 2), n_chunks):
      sd[j % 2].wait()

  return gather


def kernel(position_ids, table):
  batch, seq = position_ids.shape
  d = table.shape[1]
  ids = position_ids.reshape(-1).astype(jnp.int32)
  out = _make_gather(batch * seq, d)(ids, table)
  return out.reshape(batch, seq, d)
